# Initial kernel scaffold; baseline (speedup 1.0000x reference)
import jax
import jax.numpy as jnp
from jax.experimental import pallas as pl


def kernel(q, table):
    raise NotImplementedError("write your pallas kernel here")


# speedup vs baseline: 1.6823x; 1.6823x over previous
import jax
import jax.numpy as jnp
from jax import lax
from jax.experimental import pallas as pl
from jax.experimental.pallas import tpu as pltpu
from jax.experimental.pallas import tpu_sc as plsc

BATCH = 16384
HIST = 50
D = 64
NW = 32
TOTAL = BATCH * HIST
PER_W = TOTAL // NW
CH = 128
NCH = PER_W // CH


def _body(q_hbm, table_hbm, out_hbm, idx_v, row_v, sem):
    c = lax.axis_index("c")
    s = lax.axis_index("s")
    wid = s * 2 + c
    base = wid * PER_W
    pltpu.sync_copy(q_hbm.at[wid], idx_v)

    def step(j, carry):
        pltpu.async_copy(table_hbm.at[idx_v.at[j]], row_v, sem).wait()
        pltpu.sync_copy(row_v, out_hbm.at[pl.ds(base + j * CH, CH)])
        return carry

    lax.fori_loop(0, NCH, step, 0)


def kernel(q, table):
    qr = q.astype(jnp.int32).reshape(NW, NCH, CH)
    out = pl.kernel(
        _body,
        mesh=plsc.VectorSubcoreMesh(core_axis_name="c", subcore_axis_name="s"),
        out_type=jax.ShapeDtypeStruct((TOTAL, D), jnp.float32),
        scratch_types=[
            pltpu.VMEM((NCH, CH), jnp.int32),
            pltpu.VMEM((CH, D), jnp.float32),
            pltpu.SemaphoreType.DMA,
        ],
        compiler_params=pltpu.CompilerParams(use_tc_tiling_on_sc=False),
    )(qr, table)
    return out.reshape(BATCH, HIST, D)


# speedup vs baseline: 1.8760x; 1.1152x over previous
import jax
import jax.numpy as jnp
from jax import lax
from jax.experimental import pallas as pl
from jax.experimental.pallas import tpu as pltpu
from jax.experimental.pallas import tpu_sc as plsc

BATCH = 16384
HIST = 50
D = 64
NW = 32
TOTAL = BATCH * HIST
PER_W = TOTAL // NW
CH = 128
NCH = PER_W // CH
G = 5
PCH = G * CH
NPH = NCH // G


def _body(q_hbm, table_hbm, out_hbm, idx_v, buf_a, buf_b, gs_a, gs_b, ss_a, ss_b):
    c = lax.axis_index("c")
    s = lax.axis_index("s")
    wid = s * 2 + c
    base = wid * PER_W
    pltpu.sync_copy(q_hbm.at[wid], idx_v)

    def fire_gathers(p, buf, gsem):
        for g in range(G):
            pltpu.async_copy(table_hbm.at[idx_v.at[p * G + g]],
                             buf.at[pl.ds(g * CH, CH)], gsem)

    def drain_gathers(buf, gsem):
        for g in range(G):
            pltpu.make_async_copy(table_hbm.at[pl.ds(0, CH)],
                                  buf.at[pl.ds(g * CH, CH)], gsem).wait()

    def store_dst(p):
        return out_hbm.at[pl.ds(base + p * PCH, PCH)]

    fire_gathers(0, buf_a, gs_a)

    def two_phases(p0, carry):
        for buf, gsem, obuf, ogsem, ossem, half in (
            (buf_a, gs_a, buf_b, gs_b, ss_b, 0),
            (buf_b, gs_b, buf_a, gs_a, ss_a, 1),
        ):
            p = p0 + half

            @pl.when(p + 1 < NPH)
            def _():
                @pl.when(p >= 1)
                def _():
                    pltpu.make_async_copy(obuf, store_dst(p - 1), ossem).wait()
                fire_gathers(p + 1, obuf, ogsem)

            drain_gathers(buf, gsem)
            ssem = ss_a if half == 0 else ss_b
            pltpu.async_copy(buf, store_dst(p), ssem)
        return carry

    lax.fori_loop(0, NPH // 2, lambda i, cr: two_phases(i * 2, cr), 0)
    pltpu.make_async_copy(buf_a, store_dst(NPH - 2), ss_a).wait()
    pltpu.make_async_copy(buf_b, store_dst(NPH - 1), ss_b).wait()


def kernel(q, table):
    qr = q.astype(jnp.int32).reshape(NW, NCH, CH)
    out = pl.kernel(
        _body,
        mesh=plsc.VectorSubcoreMesh(core_axis_name="c", subcore_axis_name="s"),
        out_type=jax.ShapeDtypeStruct((TOTAL, D), jnp.float32),
        scratch_types=[
            pltpu.VMEM((NCH, CH), jnp.int32),
            pltpu.VMEM((PCH, D), jnp.float32),
            pltpu.VMEM((PCH, D), jnp.float32),
            pltpu.SemaphoreType.DMA,
            pltpu.SemaphoreType.DMA,
            pltpu.SemaphoreType.DMA,
            pltpu.SemaphoreType.DMA,
        ],
        compiler_params=pltpu.CompilerParams(use_tc_tiling_on_sc=False),
    )(qr, table)
    return out.reshape(BATCH, HIST, D)
